# Initial kernel scaffold; baseline (speedup 1.0000x reference)
import jax
import jax.numpy as jnp
from jax.experimental import pallas as pl


def kernel(x, router_dw_w, router_pw_w, router_fc_w, router_fc_b, shared_w, shared_gn_scale, shared_gn_bias, expert_w, expert_gn_scale, expert_gn_bias):
    raise NotImplementedError("write your pallas kernel here")


# speedup vs baseline: 2.5799x; 2.5799x over previous
import jax
import jax.numpy as jnp
from jax import lax
from jax.experimental import pallas as pl
from jax.experimental.pallas import tpu as pltpu

_B, _C, _O, _H, _W = 4, 96, 96, 224, 224
_E, _K, _PS, _NG = 8, 2, 8, 8
_R = 6
_HW = _H * _W
_THRESH = 0.01
_GS = _O // _NG
_T = 3584
_NT = _HW // _T
_ROWS = 8
_CHUNK = _ROWS * _W
_NP = _H // _ROWS
_PW = _W // _PS


def _silu(v):
    return v * jax.nn.sigmoid(v)


def _gram_pool_kernel(x_ref, gram_ref, xp_ref):
    i = pl.program_id(1)
    xm = x_ref[0]
    g = lax.dot_general(xm, xm, (((1,), (1,)), ((), ())),
                        preferred_element_type=jnp.float32)

    @pl.when(i == 0)
    def _():
        gram_ref[0] = g

    @pl.when(i != 0)
    def _():
        gram_ref[0] += g

    wi = lax.broadcasted_iota(jnp.int32, (_CHUNK, _PW), 0) % _W
    ji = lax.broadcasted_iota(jnp.int32, (_CHUNK, _PW), 1)
    pmat = jnp.where((wi // _PS) == ji, 1.0 / (_PS * _PS), 0.0)
    xp_ref[0, 0] = jnp.dot(xm, pmat.astype(jnp.float32),
                           preferred_element_type=jnp.float32)


def _router_kernel(xp_ref, dw_ref, pw_ref, fc_ref, fcb_ref, idx_ref, wts_ref):
    xm = xp_ref[0]
    z32 = jnp.zeros((_C, 32), jnp.float32)
    xbig = jnp.concatenate([z32, xm, z32], axis=1)
    jcol = lax.broadcasted_iota(jnp.int32, (_C, _PW * _PW), 1) % _PW
    acc = jnp.zeros((_C, _PW * _PW), jnp.float32)
    for di in (-1, 0, 1):
        for dj in (-1, 0, 1):
            k9 = (di + 1) * 3 + (dj + 1)
            base = 32 + _PW * di + dj
            term = xbig[:, base:base + _PW * _PW] * dw_ref[:, k9:k9 + 1]
            if dj == -1:
                term = jnp.where(jcol == 0, 0.0, term)
            elif dj == 1:
                term = jnp.where(jcol == _PW - 1, 0.0, term)
            acc = acc + term
    xd = _silu(acc)
    xr = _silu(jnp.dot(pw_ref[...], xd, preferred_element_type=jnp.float32))
    gap = jnp.mean(xr, axis=1, keepdims=True)
    logits = jnp.dot(fc_ref[...], gap,
                     preferred_element_type=jnp.float32) + fcb_ref[...]
    io = lax.broadcasted_iota(jnp.int32, (_E, 1), 0)
    m1 = jnp.max(logits)
    i1 = jnp.min(jnp.where(logits == m1, io, _E))
    m2 = jnp.max(jnp.where(io == i1, -1e30, logits))
    i2 = jnp.min(jnp.where((logits == m2) & (io != i1), io, _E))
    e = jnp.exp(m2 - m1)
    w1 = 1.0 / (1.0 + e)
    w2 = e / (1.0 + e)
    w1 = jnp.where(w1 >= _THRESH, w1, 0.0)
    w2 = jnp.where(w2 >= _THRESH, w2, 0.0)
    lane = lax.broadcasted_iota(jnp.int32, (1, 1, 8), 2)
    idx_ref[...] = jnp.where(lane == 0, i1,
                             jnp.where(lane == 1, i2, 0)).astype(jnp.int32)
    wts_ref[...] = jnp.where(lane == 0, w1, jnp.where(lane == 1, w2, 0.0))


def _stats_kernel(idx_pref, gram_ref, s_ref, expw_ref, expsb_ref,
                  shw_ref, shsb_ref, wts_ref, mcat_ref, aff_ref):
    j = pl.program_id(1)
    is_sh = j == 0
    wu = jnp.where(is_sh, shw_ref[...], expw_ref[0])
    sb = jnp.where(is_sh, shsb_ref[...], expsb_ref[0])
    g = gram_ref[0]
    s = s_ref[0]
    m = jnp.dot(wu, s, preferred_element_type=jnp.float32) / _HW
    t = jnp.dot(wu, g, preferred_element_type=jnp.float32)
    q = jnp.sum(t * wu, axis=1, keepdims=True) / _HW
    gi = lax.broadcasted_iota(jnp.int32, (_O, _O), 0) // _GS
    gj = lax.broadcasted_iota(jnp.int32, (_O, _O), 1) // _GS
    p = jnp.where(gi == gj, 1.0 / _GS, 0.0)
    mu = jnp.dot(p, m, preferred_element_type=jnp.float32)
    var = jnp.dot(p, q, preferred_element_type=jnp.float32) - mu * mu
    rsig = lax.rsqrt(var + 1e-5)
    a = rsig * sb[:, 0:1]
    bv = sb[:, 1:2] - mu * a
    lane8 = lax.broadcasted_iota(jnp.int32, (1, 8), 1)
    wk = jnp.sum(jnp.where(lane8 == (j - 1), wts_ref[0], 0.0))
    c = jnp.where(is_sh, 1.0, wk)
    cc = jnp.zeros((_O, 1), jnp.float32) + c
    mcat_ref[0, 0] = wu
    aff_ref[0, 0] = jnp.concatenate(
        [a, bv, cc, jnp.zeros((_O, 5), jnp.float32)], axis=1)


def _main_kernel(x_ref, mcat_ref, aff_ref, out_ref):
    xt = x_ref[0]
    m = mcat_ref[0].reshape(3 * _O, _C)
    z = jnp.dot(m, xt, preferred_element_type=jnp.float32)
    acc = None
    for j in range(3):
        zj = z[_O * j:_O * (j + 1)]
        af = aff_ref[0, j]
        tj = af[:, 0:1] * zj + af[:, 1:2]
        oj = af[:, 2:3] * (tj * jax.nn.sigmoid(tj))
        acc = oj if acc is None else acc + oj
    out_ref[0] = acc


def kernel(x, router_dw_w, router_pw_w, router_fc_w, router_fc_b,
           shared_w, shared_gn_scale, shared_gn_bias,
           expert_w, expert_gn_scale, expert_gn_bias):
    x3 = x.reshape(_B, _C, _HW)

    gram, xp4 = pl.pallas_call(
        _gram_pool_kernel,
        grid=(_B, _NP),
        in_specs=[pl.BlockSpec((1, _C, _CHUNK), lambda b, i: (b, 0, i))],
        out_specs=[pl.BlockSpec((1, _C, _C), lambda b, i: (b, 0, 0)),
                   pl.BlockSpec((1, 1, _C, _PW), lambda b, i: (b, i, 0, 0))],
        out_shape=[jax.ShapeDtypeStruct((_B, _C, _C), jnp.float32),
                   jax.ShapeDtypeStruct((_B, _NP, _C, _PW), jnp.float32)],
    )(x3)

    xp3 = xp4.transpose(0, 2, 1, 3).reshape(_B, _C, _PW * _PW)
    s_col = (xp4.sum(axis=(1, 3)) * (_PS * _PS)).reshape(_B, _C, 1)
    dw9 = router_dw_w.reshape(_C, 9)
    pw8 = jnp.zeros((8, _C), jnp.float32).at[:_R].set(router_pw_w)
    fc8 = jnp.zeros((_E, 8), jnp.float32).at[:, :_R].set(router_fc_w)
    fcb = router_fc_b.reshape(_E, 1)

    idx_o, wts_o = pl.pallas_call(
        _router_kernel,
        grid=(_B,),
        in_specs=[pl.BlockSpec((1, _C, _PW * _PW), lambda b: (b, 0, 0)),
                  pl.BlockSpec((_C, 9), lambda b: (0, 0)),
                  pl.BlockSpec((8, _C), lambda b: (0, 0)),
                  pl.BlockSpec((_E, 8), lambda b: (0, 0)),
                  pl.BlockSpec((_E, 1), lambda b: (0, 0))],
        out_specs=[pl.BlockSpec((1, 1, 8), lambda b: (b, 0, 0)),
                   pl.BlockSpec((1, 1, 8), lambda b: (b, 0, 0))],
        out_shape=[jax.ShapeDtypeStruct((_B, 1, 8), jnp.int32),
                   jax.ShapeDtypeStruct((_B, 1, 8), jnp.float32)],
    )(xp3, dw9, pw8, fc8, fcb)

    idx_flat = idx_o[:, 0, :_K].reshape(-1)
    expsb = jnp.stack([expert_gn_scale, expert_gn_bias], axis=-1)
    shsb = jnp.stack([shared_gn_scale, shared_gn_bias], axis=-1)

    grid_spec = pltpu.PrefetchScalarGridSpec(
        num_scalar_prefetch=1,
        grid=(_B, 3),
        in_specs=[
            pl.BlockSpec((1, _C, _C), lambda b, j, ref: (b, 0, 0)),
            pl.BlockSpec((1, _C, 1), lambda b, j, ref: (b, 0, 0)),
            pl.BlockSpec((1, _O, _C),
                         lambda b, j, ref: (ref[b * _K + jnp.maximum(j - 1, 0)], 0, 0)),
            pl.BlockSpec((1, _O, 2),
                         lambda b, j, ref: (ref[b * _K + jnp.maximum(j - 1, 0)], 0, 0)),
            pl.BlockSpec((_O, _C), lambda b, j, ref: (0, 0)),
            pl.BlockSpec((_O, 2), lambda b, j, ref: (0, 0)),
            pl.BlockSpec((1, 1, 8), lambda b, j, ref: (b, 0, 0)),
        ],
        out_specs=[
            pl.BlockSpec((1, 1, _O, _C), lambda b, j, ref: (b, j, 0, 0)),
            pl.BlockSpec((1, 1, _O, 8), lambda b, j, ref: (b, j, 0, 0)),
        ],
    )
    mcat, aff = pl.pallas_call(
        _stats_kernel,
        grid_spec=grid_spec,
        out_shape=[jax.ShapeDtypeStruct((_B, 3, _O, _C), jnp.float32),
                   jax.ShapeDtypeStruct((_B, 3, _O, 8), jnp.float32)],
    )(idx_flat, gram, s_col, expert_w, expsb, shared_w, shsb, wts_o)

    out3 = pl.pallas_call(
        _main_kernel,
        grid=(_B, _NT),
        in_specs=[pl.BlockSpec((1, _C, _T), lambda b, t: (b, 0, t)),
                  pl.BlockSpec((1, 3, _O, _C), lambda b, t: (b, 0, 0, 0)),
                  pl.BlockSpec((1, 3, _O, 8), lambda b, t: (b, 0, 0, 0))],
        out_specs=pl.BlockSpec((1, _O, _T), lambda b, t: (b, 0, t)),
        out_shape=jax.ShapeDtypeStruct((_B, _O, _HW), jnp.float32),
    )(x3, mcat, aff)

    return out3.reshape(_B, _O, _H, _W)


# speedup vs baseline: 2.5877x; 1.0030x over previous
import jax
import jax.numpy as jnp
from jax import lax
from jax.experimental import pallas as pl
from jax.experimental.pallas import tpu as pltpu

_B, _C, _O, _H, _W = 4, 96, 96, 224, 224
_E, _K, _PS, _NG = 8, 2, 8, 8
_R = 6
_HW = _H * _W
_THRESH = 0.01
_GS = _O // _NG
_T = 3584
_NT = _HW // _T
_ROWS = 8
_CHUNK = _ROWS * _W
_NP = _H // _ROWS
_PW = _W // _PS


def _silu(v):
    return v * jax.nn.sigmoid(v)


def _gram_pool_kernel(x_ref, gram_ref, xp_ref):
    i = pl.program_id(1)
    xm = x_ref[0]
    xmb = xm.astype(jnp.bfloat16)
    g = lax.dot_general(xmb, xmb, (((1,), (1,)), ((), ())),
                        preferred_element_type=jnp.float32)

    @pl.when(i == 0)
    def _():
        gram_ref[0] = g

    @pl.when(i != 0)
    def _():
        gram_ref[0] += g

    wi = lax.broadcasted_iota(jnp.int32, (_CHUNK, _PW), 0) % _W
    ji = lax.broadcasted_iota(jnp.int32, (_CHUNK, _PW), 1)
    pmat = jnp.where((wi // _PS) == ji, 1.0 / (_PS * _PS), 0.0)
    xp_ref[0, 0] = jnp.dot(xm, pmat.astype(jnp.float32),
                           preferred_element_type=jnp.float32)


def _router_kernel(xp_ref, dw_ref, pw_ref, fc_ref, fcb_ref, idx_ref, wts_ref):
    xm = xp_ref[0]
    z32 = jnp.zeros((_C, 32), jnp.float32)
    xbig = jnp.concatenate([z32, xm, z32], axis=1)
    jcol = lax.broadcasted_iota(jnp.int32, (_C, _PW * _PW), 1) % _PW
    acc = jnp.zeros((_C, _PW * _PW), jnp.float32)
    for di in (-1, 0, 1):
        for dj in (-1, 0, 1):
            k9 = (di + 1) * 3 + (dj + 1)
            base = 32 + _PW * di + dj
            term = xbig[:, base:base + _PW * _PW] * dw_ref[:, k9:k9 + 1]
            if dj == -1:
                term = jnp.where(jcol == 0, 0.0, term)
            elif dj == 1:
                term = jnp.where(jcol == _PW - 1, 0.0, term)
            acc = acc + term
    xd = _silu(acc)
    xr = _silu(jnp.dot(pw_ref[...], xd, preferred_element_type=jnp.float32))
    gap = jnp.mean(xr, axis=1, keepdims=True)
    logits = jnp.dot(fc_ref[...], gap,
                     preferred_element_type=jnp.float32) + fcb_ref[...]
    io = lax.broadcasted_iota(jnp.int32, (_E, 1), 0)
    m1 = jnp.max(logits)
    i1 = jnp.min(jnp.where(logits == m1, io, _E))
    m2 = jnp.max(jnp.where(io == i1, -1e30, logits))
    i2 = jnp.min(jnp.where((logits == m2) & (io != i1), io, _E))
    e = jnp.exp(m2 - m1)
    w1 = 1.0 / (1.0 + e)
    w2 = e / (1.0 + e)
    w1 = jnp.where(w1 >= _THRESH, w1, 0.0)
    w2 = jnp.where(w2 >= _THRESH, w2, 0.0)
    lane = lax.broadcasted_iota(jnp.int32, (1, 1, 8), 2)
    idx_ref[...] = jnp.where(lane == 0, i1,
                             jnp.where(lane == 1, i2, 0)).astype(jnp.int32)
    wts_ref[...] = jnp.where(lane == 0, w1, jnp.where(lane == 1, w2, 0.0))


def _stats_kernel(idx_pref, gram_ref, s_ref, expw_ref, expsb_ref,
                  shw_ref, shsb_ref, wts_ref, mcat_ref, aff_ref):
    j = pl.program_id(1)
    is_sh = j == 0
    wu = jnp.where(is_sh, shw_ref[...], expw_ref[0])
    sb = jnp.where(is_sh, shsb_ref[...], expsb_ref[0])
    g = gram_ref[0]
    s = s_ref[0]
    m = jnp.dot(wu, s, preferred_element_type=jnp.float32) / _HW
    t = jnp.dot(wu, g, preferred_element_type=jnp.float32)
    q = jnp.sum(t * wu, axis=1, keepdims=True) / _HW
    gi = lax.broadcasted_iota(jnp.int32, (_O, _O), 0) // _GS
    gj = lax.broadcasted_iota(jnp.int32, (_O, _O), 1) // _GS
    p = jnp.where(gi == gj, 1.0 / _GS, 0.0)
    mu = jnp.dot(p, m, preferred_element_type=jnp.float32)
    var = jnp.dot(p, q, preferred_element_type=jnp.float32) - mu * mu
    rsig = lax.rsqrt(var + 1e-5)
    a = rsig * sb[:, 0:1]
    bv = sb[:, 1:2] - mu * a
    lane8 = lax.broadcasted_iota(jnp.int32, (1, 8), 1)
    wk = jnp.sum(jnp.where(lane8 == (j - 1), wts_ref[0], 0.0))
    c = jnp.where(is_sh, 1.0, wk)
    cc = jnp.zeros((_O, 1), jnp.float32) + c
    mcat_ref[0, 0] = wu
    aff_ref[0, 0] = jnp.concatenate(
        [a, bv, cc, jnp.zeros((_O, 5), jnp.float32)], axis=1)


def _main_kernel(x_ref, mcat_ref, aff_ref, out_ref):
    xt = x_ref[0]
    m = mcat_ref[0].reshape(3 * _O, _C)
    z = jnp.dot(m.astype(jnp.bfloat16), xt.astype(jnp.bfloat16),
                preferred_element_type=jnp.float32)
    acc = None
    for j in range(3):
        zj = z[_O * j:_O * (j + 1)]
        af = aff_ref[0, j]
        tj = af[:, 0:1] * zj + af[:, 1:2]
        oj = af[:, 2:3] * (tj * jax.nn.sigmoid(tj))
        acc = oj if acc is None else acc + oj
    out_ref[0] = acc


def kernel(x, router_dw_w, router_pw_w, router_fc_w, router_fc_b,
           shared_w, shared_gn_scale, shared_gn_bias,
           expert_w, expert_gn_scale, expert_gn_bias):
    x3 = x.reshape(_B, _C, _HW)

    gram, xp4 = pl.pallas_call(
        _gram_pool_kernel,
        grid=(_B, _NP),
        in_specs=[pl.BlockSpec((1, _C, _CHUNK), lambda b, i: (b, 0, i))],
        out_specs=[pl.BlockSpec((1, _C, _C), lambda b, i: (b, 0, 0)),
                   pl.BlockSpec((1, 1, _C, _PW), lambda b, i: (b, i, 0, 0))],
        out_shape=[jax.ShapeDtypeStruct((_B, _C, _C), jnp.float32),
                   jax.ShapeDtypeStruct((_B, _NP, _C, _PW), jnp.float32)],
    )(x3)

    xp3 = xp4.transpose(0, 2, 1, 3).reshape(_B, _C, _PW * _PW)
    s_col = (xp4.sum(axis=(1, 3)) * (_PS * _PS)).reshape(_B, _C, 1)
    dw9 = router_dw_w.reshape(_C, 9)
    pw8 = jnp.zeros((8, _C), jnp.float32).at[:_R].set(router_pw_w)
    fc8 = jnp.zeros((_E, 8), jnp.float32).at[:, :_R].set(router_fc_w)
    fcb = router_fc_b.reshape(_E, 1)

    idx_o, wts_o = pl.pallas_call(
        _router_kernel,
        grid=(_B,),
        in_specs=[pl.BlockSpec((1, _C, _PW * _PW), lambda b: (b, 0, 0)),
                  pl.BlockSpec((_C, 9), lambda b: (0, 0)),
                  pl.BlockSpec((8, _C), lambda b: (0, 0)),
                  pl.BlockSpec((_E, 8), lambda b: (0, 0)),
                  pl.BlockSpec((_E, 1), lambda b: (0, 0))],
        out_specs=[pl.BlockSpec((1, 1, 8), lambda b: (b, 0, 0)),
                   pl.BlockSpec((1, 1, 8), lambda b: (b, 0, 0))],
        out_shape=[jax.ShapeDtypeStruct((_B, 1, 8), jnp.int32),
                   jax.ShapeDtypeStruct((_B, 1, 8), jnp.float32)],
    )(xp3, dw9, pw8, fc8, fcb)

    idx_flat = idx_o[:, 0, :_K].reshape(-1)
    expsb = jnp.stack([expert_gn_scale, expert_gn_bias], axis=-1)
    shsb = jnp.stack([shared_gn_scale, shared_gn_bias], axis=-1)

    grid_spec = pltpu.PrefetchScalarGridSpec(
        num_scalar_prefetch=1,
        grid=(_B, 3),
        in_specs=[
            pl.BlockSpec((1, _C, _C), lambda b, j, ref: (b, 0, 0)),
            pl.BlockSpec((1, _C, 1), lambda b, j, ref: (b, 0, 0)),
            pl.BlockSpec((1, _O, _C),
                         lambda b, j, ref: (ref[b * _K + jnp.maximum(j - 1, 0)], 0, 0)),
            pl.BlockSpec((1, _O, 2),
                         lambda b, j, ref: (ref[b * _K + jnp.maximum(j - 1, 0)], 0, 0)),
            pl.BlockSpec((_O, _C), lambda b, j, ref: (0, 0)),
            pl.BlockSpec((_O, 2), lambda b, j, ref: (0, 0)),
            pl.BlockSpec((1, 1, 8), lambda b, j, ref: (b, 0, 0)),
        ],
        out_specs=[
            pl.BlockSpec((1, 1, _O, _C), lambda b, j, ref: (b, j, 0, 0)),
            pl.BlockSpec((1, 1, _O, 8), lambda b, j, ref: (b, j, 0, 0)),
        ],
    )
    mcat, aff = pl.pallas_call(
        _stats_kernel,
        grid_spec=grid_spec,
        out_shape=[jax.ShapeDtypeStruct((_B, 3, _O, _C), jnp.float32),
                   jax.ShapeDtypeStruct((_B, 3, _O, 8), jnp.float32)],
    )(idx_flat, gram, s_col, expert_w, expsb, shared_w, shsb, wts_o)

    out3 = pl.pallas_call(
        _main_kernel,
        grid=(_B, _NT),
        in_specs=[pl.BlockSpec((1, _C, _T), lambda b, t: (b, 0, t)),
                  pl.BlockSpec((1, 3, _O, _C), lambda b, t: (b, 0, 0, 0)),
                  pl.BlockSpec((1, 3, _O, 8), lambda b, t: (b, 0, 0, 0))],
        out_specs=pl.BlockSpec((1, _O, _T), lambda b, t: (b, 0, t)),
        out_shape=jax.ShapeDtypeStruct((_B, _O, _HW), jnp.float32),
    )(x3, mcat, aff)

    return out3.reshape(_B, _O, _H, _W)


# speedup vs baseline: 3.2091x; 1.2401x over previous
import jax
import jax.numpy as jnp
from jax import lax
from jax.experimental import pallas as pl
from jax.experimental.pallas import tpu as pltpu

_B, _C, _O, _H, _W = 4, 96, 96, 224, 224
_E, _K, _PS, _NG = 8, 2, 8, 8
_R = 6
_HW = _H * _W
_THRESH = 0.01
_GS = _O // _NG
_T = 3584
_NT = _HW // _T
_ROWS = 16
_CHUNK = _ROWS * _W
_NP = _H // _ROWS
_PW = _W // _PS
_PR = _ROWS // _PS


def _silu(v):
    return v * jax.nn.sigmoid(v)


def _gram_pool_kernel(x_ref, pmat_ref, gram_ref, xp_ref):
    i = pl.program_id(1)
    xm = x_ref[0]
    xmb = xm.astype(jnp.bfloat16)
    g = lax.dot_general(xmb, xmb, (((1,), (1,)), ((), ())),
                        preferred_element_type=jnp.float32)

    @pl.when(i == 0)
    def _():
        gram_ref[0] = g

    @pl.when(i != 0)
    def _():
        gram_ref[0] += g

    pooled = jnp.dot(xm, pmat_ref[...], preferred_element_type=jnp.float32)
    xp_ref[0, 0] = pooled[:, :_PW]
    xp_ref[0, 1] = pooled[:, _PW:]


def _router_kernel(xp_ref, dw_ref, pw_ref, fc_ref, fcb_ref, idx_ref, wts_ref):
    xm = xp_ref[0]
    z32 = jnp.zeros((_C, 32), jnp.float32)
    xbig = jnp.concatenate([z32, xm, z32], axis=1)
    jcol = lax.broadcasted_iota(jnp.int32, (_C, _PW * _PW), 1) % _PW
    acc = jnp.zeros((_C, _PW * _PW), jnp.float32)
    for di in (-1, 0, 1):
        for dj in (-1, 0, 1):
            k9 = (di + 1) * 3 + (dj + 1)
            base = 32 + _PW * di + dj
            term = xbig[:, base:base + _PW * _PW] * dw_ref[:, k9:k9 + 1]
            if dj == -1:
                term = jnp.where(jcol == 0, 0.0, term)
            elif dj == 1:
                term = jnp.where(jcol == _PW - 1, 0.0, term)
            acc = acc + term
    xd = _silu(acc)
    xr = _silu(jnp.dot(pw_ref[...], xd, preferred_element_type=jnp.float32))
    gap = jnp.mean(xr, axis=1, keepdims=True)
    logits = jnp.dot(fc_ref[...], gap,
                     preferred_element_type=jnp.float32) + fcb_ref[...]
    io = lax.broadcasted_iota(jnp.int32, (_E, 1), 0)
    m1 = jnp.max(logits)
    i1 = jnp.min(jnp.where(logits == m1, io, _E))
    m2 = jnp.max(jnp.where(io == i1, -1e30, logits))
    i2 = jnp.min(jnp.where((logits == m2) & (io != i1), io, _E))
    e = jnp.exp(m2 - m1)
    w1 = 1.0 / (1.0 + e)
    w2 = e / (1.0 + e)
    w1 = jnp.where(w1 >= _THRESH, w1, 0.0)
    w2 = jnp.where(w2 >= _THRESH, w2, 0.0)
    lane = lax.broadcasted_iota(jnp.int32, (1, 1, 8), 2)
    idx_ref[...] = jnp.where(lane == 0, i1,
                             jnp.where(lane == 1, i2, 0)).astype(jnp.int32)
    wts_ref[...] = jnp.where(lane == 0, w1, jnp.where(lane == 1, w2, 0.0))


def _stats_kernel(idx_pref, gram_ref, s_ref, expw_ref, expsb_ref,
                  shw_ref, shsb_ref, wts_ref, mcat_ref, aff_ref):
    j = pl.program_id(1)
    is_sh = j == 0
    wu = jnp.where(is_sh, shw_ref[...], expw_ref[0])
    sb = jnp.where(is_sh, shsb_ref[...], expsb_ref[0])
    g = gram_ref[0]
    s = s_ref[0]
    m = jnp.dot(wu, s, preferred_element_type=jnp.float32) / _HW
    t = jnp.dot(wu, g, preferred_element_type=jnp.float32)
    q = jnp.sum(t * wu, axis=1, keepdims=True) / _HW
    gi = lax.broadcasted_iota(jnp.int32, (_O, _O), 0) // _GS
    gj = lax.broadcasted_iota(jnp.int32, (_O, _O), 1) // _GS
    p = jnp.where(gi == gj, 1.0 / _GS, 0.0)
    mu = jnp.dot(p, m, preferred_element_type=jnp.float32)
    var = jnp.dot(p, q, preferred_element_type=jnp.float32) - mu * mu
    rsig = lax.rsqrt(var + 1e-5)
    a = rsig * sb[:, 0:1]
    bv = sb[:, 1:2] - mu * a
    lane8 = lax.broadcasted_iota(jnp.int32, (1, 8), 1)
    wk = jnp.sum(jnp.where(lane8 == (j - 1), wts_ref[0], 0.0))
    c = jnp.where(is_sh, 1.0, wk)
    cc = jnp.zeros((_O, 1), jnp.float32) + c
    mcat_ref[0, 0] = wu * a
    aff_ref[0, 0] = jnp.concatenate(
        [bv, cc, jnp.zeros((_O, 6), jnp.float32)], axis=1)


def _main_kernel(x_ref, mcat_ref, aff_ref, out_ref):
    xt = x_ref[0]
    m = mcat_ref[0].reshape(3 * _O, _C)
    z = jnp.dot(m.astype(jnp.bfloat16), xt.astype(jnp.bfloat16),
                preferred_element_type=jnp.float32)
    acc = None
    for j in range(3):
        zj = z[_O * j:_O * (j + 1)]
        af = aff_ref[0, j]
        tj = zj + af[:, 0:1]
        oj = af[:, 1:2] * (tj * jax.nn.sigmoid(tj))
        acc = oj if acc is None else acc + oj
    out_ref[0] = acc


def kernel(x, router_dw_w, router_pw_w, router_fc_w, router_fc_b,
           shared_w, shared_gn_scale, shared_gn_bias,
           expert_w, expert_gn_scale, expert_gn_bias):
    x3 = x.reshape(_B, _C, _HW)

    ridx = jnp.arange(_CHUNK) // _W
    widx = jnp.arange(_CHUNK) % _W
    pcol = (ridx // _PS) * _PW + widx // _PS
    pmat = ((pcol[:, None] == jnp.arange(_PR * _PW)[None, :])
            .astype(jnp.float32) / (_PS * _PS))

    gram, xp4 = pl.pallas_call(
        _gram_pool_kernel,
        grid=(_B, _NP),
        in_specs=[pl.BlockSpec((1, _C, _CHUNK), lambda b, i: (b, 0, i)),
                  pl.BlockSpec((_CHUNK, _PR * _PW), lambda b, i: (0, 0))],
        out_specs=[pl.BlockSpec((1, _C, _C), lambda b, i: (b, 0, 0)),
                   pl.BlockSpec((1, _PR, _C, _PW), lambda b, i: (b, i, 0, 0))],
        out_shape=[jax.ShapeDtypeStruct((_B, _C, _C), jnp.float32),
                   jax.ShapeDtypeStruct((_B, _NP * _PR, _C, _PW), jnp.float32)],
    )(x3, pmat)

    xp3 = xp4.transpose(0, 2, 1, 3).reshape(_B, _C, _PW * _PW)
    s_col = (xp4.sum(axis=(1, 3)) * (_PS * _PS)).reshape(_B, _C, 1)
    dw9 = router_dw_w.reshape(_C, 9)
    pw8 = jnp.zeros((8, _C), jnp.float32).at[:_R].set(router_pw_w)
    fc8 = jnp.zeros((_E, 8), jnp.float32).at[:, :_R].set(router_fc_w)
    fcb = router_fc_b.reshape(_E, 1)

    idx_o, wts_o = pl.pallas_call(
        _router_kernel,
        grid=(_B,),
        in_specs=[pl.BlockSpec((1, _C, _PW * _PW), lambda b: (b, 0, 0)),
                  pl.BlockSpec((_C, 9), lambda b: (0, 0)),
                  pl.BlockSpec((8, _C), lambda b: (0, 0)),
                  pl.BlockSpec((_E, 8), lambda b: (0, 0)),
                  pl.BlockSpec((_E, 1), lambda b: (0, 0))],
        out_specs=[pl.BlockSpec((1, 1, 8), lambda b: (b, 0, 0)),
                   pl.BlockSpec((1, 1, 8), lambda b: (b, 0, 0))],
        out_shape=[jax.ShapeDtypeStruct((_B, 1, 8), jnp.int32),
                   jax.ShapeDtypeStruct((_B, 1, 8), jnp.float32)],
    )(xp3, dw9, pw8, fc8, fcb)

    idx_flat = idx_o[:, 0, :_K].reshape(-1)
    expsb = jnp.stack([expert_gn_scale, expert_gn_bias], axis=-1)
    shsb = jnp.stack([shared_gn_scale, shared_gn_bias], axis=-1)

    grid_spec = pltpu.PrefetchScalarGridSpec(
        num_scalar_prefetch=1,
        grid=(_B, 3),
        in_specs=[
            pl.BlockSpec((1, _C, _C), lambda b, j, ref: (b, 0, 0)),
            pl.BlockSpec((1, _C, 1), lambda b, j, ref: (b, 0, 0)),
            pl.BlockSpec((1, _O, _C),
                         lambda b, j, ref: (ref[b * _K + jnp.maximum(j - 1, 0)], 0, 0)),
            pl.BlockSpec((1, _O, 2),
                         lambda b, j, ref: (ref[b * _K + jnp.maximum(j - 1, 0)], 0, 0)),
            pl.BlockSpec((_O, _C), lambda b, j, ref: (0, 0)),
            pl.BlockSpec((_O, 2), lambda b, j, ref: (0, 0)),
            pl.BlockSpec((1, 1, 8), lambda b, j, ref: (b, 0, 0)),
        ],
        out_specs=[
            pl.BlockSpec((1, 1, _O, _C), lambda b, j, ref: (b, j, 0, 0)),
            pl.BlockSpec((1, 1, _O, 8), lambda b, j, ref: (b, j, 0, 0)),
        ],
    )
    mcat, aff = pl.pallas_call(
        _stats_kernel,
        grid_spec=grid_spec,
        out_shape=[jax.ShapeDtypeStruct((_B, 3, _O, _C), jnp.float32),
                   jax.ShapeDtypeStruct((_B, 3, _O, 8), jnp.float32)],
    )(idx_flat, gram, s_col, expert_w, expsb, shared_w, shsb, wts_o)

    out3 = pl.pallas_call(
        _main_kernel,
        grid=(_B, _NT),
        in_specs=[pl.BlockSpec((1, _C, _T), lambda b, t: (b, 0, t)),
                  pl.BlockSpec((1, 3, _O, _C), lambda b, t: (b, 0, 0, 0)),
                  pl.BlockSpec((1, 3, _O, 8), lambda b, t: (b, 0, 0, 0))],
        out_specs=pl.BlockSpec((1, _O, _T), lambda b, t: (b, 0, t)),
        out_shape=jax.ShapeDtypeStruct((_B, _O, _HW), jnp.float32),
    )(x3, mcat, aff)

    return out3.reshape(_B, _O, _H, _W)


# speedup vs baseline: 3.4556x; 1.0768x over previous
import jax
import jax.numpy as jnp
from jax import lax
from jax.experimental import pallas as pl
from jax.experimental.pallas import tpu as pltpu

_B, _C, _O, _H, _W = 4, 96, 96, 224, 224
_E, _K, _PS, _NG = 8, 2, 8, 8
_R = 6
_HW = _H * _W
_THRESH = 0.01
_GS = _O // _NG
_T = 7168
_NT = _HW // _T
_ROWS = 32
_CHUNK = _ROWS * _W
_NP = _H // _ROWS
_PW = _W // _PS
_PR = _ROWS // _PS


def _silu(v):
    return v * jax.nn.sigmoid(v)


def _gram_pool_kernel(x_ref, pmat_ref, gram_ref, xp_ref):
    i = pl.program_id(1)
    xm = x_ref[0]
    xmb = xm.astype(jnp.bfloat16)
    g = lax.dot_general(xmb, xmb, (((1,), (1,)), ((), ())),
                        preferred_element_type=jnp.float32)

    @pl.when(i == 0)
    def _():
        gram_ref[0] = g

    @pl.when(i != 0)
    def _():
        gram_ref[0] += g

    pooled = jnp.dot(xm, pmat_ref[...], preferred_element_type=jnp.float32)
    for k in range(_PR):
        xp_ref[0, k] = pooled[:, k * _PW:(k + 1) * _PW]


def _router_kernel(xp_ref, dw_ref, pw_ref, fc_ref, fcb_ref, idx_ref, wts_ref):
    xm = xp_ref[0]
    z32 = jnp.zeros((_C, 32), jnp.float32)
    xbig = jnp.concatenate([z32, xm, z32], axis=1)
    jcol = lax.broadcasted_iota(jnp.int32, (_C, _PW * _PW), 1) % _PW
    acc = jnp.zeros((_C, _PW * _PW), jnp.float32)
    for di in (-1, 0, 1):
        for dj in (-1, 0, 1):
            k9 = (di + 1) * 3 + (dj + 1)
            base = 32 + _PW * di + dj
            term = xbig[:, base:base + _PW * _PW] * dw_ref[:, k9:k9 + 1]
            if dj == -1:
                term = jnp.where(jcol == 0, 0.0, term)
            elif dj == 1:
                term = jnp.where(jcol == _PW - 1, 0.0, term)
            acc = acc + term
    xd = _silu(acc)
    xr = _silu(jnp.dot(pw_ref[...], xd, preferred_element_type=jnp.float32))
    gap = jnp.mean(xr, axis=1, keepdims=True)
    logits = jnp.dot(fc_ref[...], gap,
                     preferred_element_type=jnp.float32) + fcb_ref[...]
    io = lax.broadcasted_iota(jnp.int32, (_E, 1), 0)
    m1 = jnp.max(logits)
    i1 = jnp.min(jnp.where(logits == m1, io, _E))
    m2 = jnp.max(jnp.where(io == i1, -1e30, logits))
    i2 = jnp.min(jnp.where((logits == m2) & (io != i1), io, _E))
    e = jnp.exp(m2 - m1)
    w1 = 1.0 / (1.0 + e)
    w2 = e / (1.0 + e)
    w1 = jnp.where(w1 >= _THRESH, w1, 0.0)
    w2 = jnp.where(w2 >= _THRESH, w2, 0.0)
    lane = lax.broadcasted_iota(jnp.int32, (1, 1, 8), 2)
    idx_ref[...] = jnp.where(lane == 0, i1,
                             jnp.where(lane == 1, i2, 0)).astype(jnp.int32)
    wts_ref[...] = jnp.where(lane == 0, w1, jnp.where(lane == 1, w2, 0.0))


def _stats_kernel(idx_pref, gram_ref, s_ref, expw_ref, expsb_ref,
                  shw_ref, shsb_ref, wts_ref, mcat_ref, aff_ref):
    j = pl.program_id(1)
    is_sh = j == 0
    wu = jnp.where(is_sh, shw_ref[...], expw_ref[0])
    sb = jnp.where(is_sh, shsb_ref[...], expsb_ref[0])
    g = gram_ref[0]
    s = s_ref[0]
    m = jnp.dot(wu, s, preferred_element_type=jnp.float32) / _HW
    t = jnp.dot(wu, g, preferred_element_type=jnp.float32)
    q = jnp.sum(t * wu, axis=1, keepdims=True) / _HW
    gi = lax.broadcasted_iota(jnp.int32, (_O, _O), 0) // _GS
    gj = lax.broadcasted_iota(jnp.int32, (_O, _O), 1) // _GS
    p = jnp.where(gi == gj, 1.0 / _GS, 0.0)
    mu = jnp.dot(p, m, preferred_element_type=jnp.float32)
    var = jnp.dot(p, q, preferred_element_type=jnp.float32) - mu * mu
    rsig = lax.rsqrt(var + 1e-5)
    a = rsig * sb[:, 0:1]
    bv = sb[:, 1:2] - mu * a
    lane8 = lax.broadcasted_iota(jnp.int32, (1, 8), 1)
    wk = jnp.sum(jnp.where(lane8 == (j - 1), wts_ref[0], 0.0))
    c = jnp.where(is_sh, 1.0, wk)
    cc = jnp.zeros((_O, 1), jnp.float32) + c
    mcat_ref[0, 0] = wu * a
    aff_ref[0, 0] = jnp.concatenate(
        [bv, cc, jnp.zeros((_O, 6), jnp.float32)], axis=1)


def _main_kernel(x_ref, mcat_ref, aff_ref, out_ref):
    xt = x_ref[0]
    m = mcat_ref[0].reshape(3 * _O, _C)
    z = jnp.dot(m.astype(jnp.bfloat16), xt.astype(jnp.bfloat16),
                preferred_element_type=jnp.float32)
    acc = None
    for j in range(3):
        zj = z[_O * j:_O * (j + 1)]
        af = aff_ref[0, j]
        tj = zj + af[:, 0:1]
        oj = af[:, 1:2] * (tj * jax.nn.sigmoid(tj))
        acc = oj if acc is None else acc + oj
    out_ref[0] = acc


def kernel(x, router_dw_w, router_pw_w, router_fc_w, router_fc_b,
           shared_w, shared_gn_scale, shared_gn_bias,
           expert_w, expert_gn_scale, expert_gn_bias):
    x3 = x.reshape(_B, _C, _HW)

    ridx = jnp.arange(_CHUNK) // _W
    widx = jnp.arange(_CHUNK) % _W
    pcol = (ridx // _PS) * _PW + widx // _PS
    pmat = ((pcol[:, None] == jnp.arange(_PR * _PW)[None, :])
            .astype(jnp.float32) / (_PS * _PS))

    gram, xp4 = pl.pallas_call(
        _gram_pool_kernel,
        grid=(_B, _NP),
        in_specs=[pl.BlockSpec((1, _C, _CHUNK), lambda b, i: (b, 0, i)),
                  pl.BlockSpec((_CHUNK, _PR * _PW), lambda b, i: (0, 0))],
        out_specs=[pl.BlockSpec((1, _C, _C), lambda b, i: (b, 0, 0)),
                   pl.BlockSpec((1, _PR, _C, _PW), lambda b, i: (b, i, 0, 0))],
        out_shape=[jax.ShapeDtypeStruct((_B, _C, _C), jnp.float32),
                   jax.ShapeDtypeStruct((_B, _NP * _PR, _C, _PW), jnp.float32)],
    )(x3, pmat)

    xp3 = xp4.transpose(0, 2, 1, 3).reshape(_B, _C, _PW * _PW)
    s_col = (xp4.sum(axis=(1, 3)) * (_PS * _PS)).reshape(_B, _C, 1)
    dw9 = router_dw_w.reshape(_C, 9)
    pw8 = jnp.zeros((8, _C), jnp.float32).at[:_R].set(router_pw_w)
    fc8 = jnp.zeros((_E, 8), jnp.float32).at[:, :_R].set(router_fc_w)
    fcb = router_fc_b.reshape(_E, 1)

    idx_o, wts_o = pl.pallas_call(
        _router_kernel,
        grid=(_B,),
        in_specs=[pl.BlockSpec((1, _C, _PW * _PW), lambda b: (b, 0, 0)),
                  pl.BlockSpec((_C, 9), lambda b: (0, 0)),
                  pl.BlockSpec((8, _C), lambda b: (0, 0)),
                  pl.BlockSpec((_E, 8), lambda b: (0, 0)),
                  pl.BlockSpec((_E, 1), lambda b: (0, 0))],
        out_specs=[pl.BlockSpec((1, 1, 8), lambda b: (b, 0, 0)),
                   pl.BlockSpec((1, 1, 8), lambda b: (b, 0, 0))],
        out_shape=[jax.ShapeDtypeStruct((_B, 1, 8), jnp.int32),
                   jax.ShapeDtypeStruct((_B, 1, 8), jnp.float32)],
    )(xp3, dw9, pw8, fc8, fcb)

    idx_flat = idx_o[:, 0, :_K].reshape(-1)
    expsb = jnp.stack([expert_gn_scale, expert_gn_bias], axis=-1)
    shsb = jnp.stack([shared_gn_scale, shared_gn_bias], axis=-1)

    grid_spec = pltpu.PrefetchScalarGridSpec(
        num_scalar_prefetch=1,
        grid=(_B, 3),
        in_specs=[
            pl.BlockSpec((1, _C, _C), lambda b, j, ref: (b, 0, 0)),
            pl.BlockSpec((1, _C, 1), lambda b, j, ref: (b, 0, 0)),
            pl.BlockSpec((1, _O, _C),
                         lambda b, j, ref: (ref[b * _K + jnp.maximum(j - 1, 0)], 0, 0)),
            pl.BlockSpec((1, _O, 2),
                         lambda b, j, ref: (ref[b * _K + jnp.maximum(j - 1, 0)], 0, 0)),
            pl.BlockSpec((_O, _C), lambda b, j, ref: (0, 0)),
            pl.BlockSpec((_O, 2), lambda b, j, ref: (0, 0)),
            pl.BlockSpec((1, 1, 8), lambda b, j, ref: (b, 0, 0)),
        ],
        out_specs=[
            pl.BlockSpec((1, 1, _O, _C), lambda b, j, ref: (b, j, 0, 0)),
            pl.BlockSpec((1, 1, _O, 8), lambda b, j, ref: (b, j, 0, 0)),
        ],
    )
    mcat, aff = pl.pallas_call(
        _stats_kernel,
        grid_spec=grid_spec,
        out_shape=[jax.ShapeDtypeStruct((_B, 3, _O, _C), jnp.float32),
                   jax.ShapeDtypeStruct((_B, 3, _O, 8), jnp.float32)],
    )(idx_flat, gram, s_col, expert_w, expsb, shared_w, shsb, wts_o)

    out3 = pl.pallas_call(
        _main_kernel,
        grid=(_B, _NT),
        in_specs=[pl.BlockSpec((1, _C, _T), lambda b, t: (b, 0, t)),
                  pl.BlockSpec((1, 3, _O, _C), lambda b, t: (b, 0, 0, 0)),
                  pl.BlockSpec((1, 3, _O, 8), lambda b, t: (b, 0, 0, 0))],
        out_specs=pl.BlockSpec((1, _O, _T), lambda b, t: (b, 0, t)),
        out_shape=jax.ShapeDtypeStruct((_B, _O, _HW), jnp.float32),
    )(x3, mcat, aff)

    return out3.reshape(_B, _O, _H, _W)


# speedup vs baseline: 3.8390x; 1.1110x over previous
import jax
import jax.numpy as jnp
from jax import lax
from jax.experimental import pallas as pl
from jax.experimental.pallas import tpu as pltpu

_B, _C, _O, _H, _W = 4, 96, 96, 224, 224
_E, _K, _PS, _NG = 8, 2, 8, 8
_R = 6
_HW = _H * _W
_THRESH = 0.01
_GS = _O // _NG
_T = 7168
_NT = _HW // _T
_ROWS = 16
_CHUNK = _ROWS * _W
_NP = _H // _ROWS
_PW = _W // _PS
_PR = _ROWS // _PS


def _silu(v):
    return v * jax.nn.sigmoid(v)


def _gram_pool_kernel(xa_ref, xb_ref, pmat_ref, gram_ref, xp_ref):
    pm = pmat_ref[...]
    parts = []
    g = None
    for half_ref in (xa_ref, xb_ref):
        xm = half_ref[0]
        xmb = xm.astype(jnp.bfloat16)
        gh = lax.dot_general(xmb, xmb, (((1,), (1,)), ((), ())),
                             preferred_element_type=jnp.float32)
        g = gh if g is None else g + gh
        for k in range(_NP // 2):
            sub = xm[:, k * _CHUNK:(k + 1) * _CHUNK]
            parts.append(jnp.dot(sub, pm, preferred_element_type=jnp.float32))
    gram_ref[0] = g
    xp_ref[0] = jnp.concatenate(parts, axis=1)


def _router_kernel(xp_ref, dw_ref, pw_ref, fc_ref, fcb_ref, idx_ref, wts_ref):
    xm = xp_ref[0]
    z32 = jnp.zeros((_C, 32), jnp.float32)
    xbig = jnp.concatenate([z32, xm, z32], axis=1)
    jcol = lax.broadcasted_iota(jnp.int32, (_C, _PW * _PW), 1) % _PW
    acc = jnp.zeros((_C, _PW * _PW), jnp.float32)
    for di in (-1, 0, 1):
        for dj in (-1, 0, 1):
            k9 = (di + 1) * 3 + (dj + 1)
            base = 32 + _PW * di + dj
            term = xbig[:, base:base + _PW * _PW] * dw_ref[:, k9:k9 + 1]
            if dj == -1:
                term = jnp.where(jcol == 0, 0.0, term)
            elif dj == 1:
                term = jnp.where(jcol == _PW - 1, 0.0, term)
            acc = acc + term
    xd = _silu(acc)
    xr = _silu(jnp.dot(pw_ref[...], xd, preferred_element_type=jnp.float32))
    gap = jnp.mean(xr, axis=1, keepdims=True)
    logits = jnp.dot(fc_ref[...], gap,
                     preferred_element_type=jnp.float32) + fcb_ref[...]
    io = lax.broadcasted_iota(jnp.int32, (_E, 1), 0)
    m1 = jnp.max(logits)
    i1 = jnp.min(jnp.where(logits == m1, io, _E))
    m2 = jnp.max(jnp.where(io == i1, -1e30, logits))
    i2 = jnp.min(jnp.where((logits == m2) & (io != i1), io, _E))
    e = jnp.exp(m2 - m1)
    w1 = 1.0 / (1.0 + e)
    w2 = e / (1.0 + e)
    w1 = jnp.where(w1 >= _THRESH, w1, 0.0)
    w2 = jnp.where(w2 >= _THRESH, w2, 0.0)
    lane = lax.broadcasted_iota(jnp.int32, (1, 1, 8), 2)
    idx_ref[...] = jnp.where(lane == 0, i1,
                             jnp.where(lane == 1, i2, 0)).astype(jnp.int32)
    wts_ref[...] = jnp.where(lane == 0, w1, jnp.where(lane == 1, w2, 0.0))


def _stats_kernel(idx_pref, gram_ref, s_ref, expw_ref, expsb_ref,
                  shw_ref, shsb_ref, wts_ref, mcat_ref, aff_ref):
    j = pl.program_id(1)
    is_sh = j == 0
    wu = jnp.where(is_sh, shw_ref[...], expw_ref[0])
    sb = jnp.where(is_sh, shsb_ref[...], expsb_ref[0])
    g = gram_ref[0]
    s = s_ref[0]
    m = jnp.dot(wu, s, preferred_element_type=jnp.float32) / _HW
    t = jnp.dot(wu, g, preferred_element_type=jnp.float32)
    q = jnp.sum(t * wu, axis=1, keepdims=True) / _HW
    gi = lax.broadcasted_iota(jnp.int32, (_O, _O), 0) // _GS
    gj = lax.broadcasted_iota(jnp.int32, (_O, _O), 1) // _GS
    p = jnp.where(gi == gj, 1.0 / _GS, 0.0)
    mu = jnp.dot(p, m, preferred_element_type=jnp.float32)
    var = jnp.dot(p, q, preferred_element_type=jnp.float32) - mu * mu
    rsig = lax.rsqrt(var + 1e-5)
    a = rsig * sb[:, 0:1]
    bv = sb[:, 1:2] - mu * a
    lane8 = lax.broadcasted_iota(jnp.int32, (1, 8), 1)
    wk = jnp.sum(jnp.where(lane8 == (j - 1), wts_ref[0], 0.0))
    c = jnp.where(is_sh, 1.0, wk)
    cc = jnp.zeros((_O, 1), jnp.float32) + c
    mcat_ref[0, 0] = wu * a
    aff_ref[0, 0] = jnp.concatenate(
        [bv, cc, jnp.zeros((_O, 6), jnp.float32)], axis=1)


def _main_kernel(x_ref, mcat_ref, aff_ref, out_ref):
    xt = x_ref[0]
    m = mcat_ref[0].reshape(3 * _O, _C)
    z = jnp.dot(m.astype(jnp.bfloat16), xt.astype(jnp.bfloat16),
                preferred_element_type=jnp.float32)
    acc = None
    for j in range(3):
        zj = z[_O * j:_O * (j + 1)]
        af = aff_ref[0, j]
        tj = zj + af[:, 0:1]
        oj = af[:, 1:2] * (tj * jax.nn.sigmoid(tj))
        acc = oj if acc is None else acc + oj
    out_ref[0] = acc


def kernel(x, router_dw_w, router_pw_w, router_fc_w, router_fc_b,
           shared_w, shared_gn_scale, shared_gn_bias,
           expert_w, expert_gn_scale, expert_gn_bias):
    x3 = x.reshape(_B, _C, _HW)

    ridx = jnp.arange(_CHUNK) // _W
    widx = jnp.arange(_CHUNK) % _W
    pcol = (ridx // _PS) * _PW + widx // _PS
    pmat = ((pcol[:, None] == jnp.arange(_PR * _PW)[None, :])
            .astype(jnp.float32) / (_PS * _PS))

    gram, xp3 = pl.pallas_call(
        _gram_pool_kernel,
        grid=(_B,),
        in_specs=[pl.BlockSpec((1, _C, _HW // 2), lambda b: (b, 0, 0)),
                  pl.BlockSpec((1, _C, _HW // 2), lambda b: (b, 0, 1)),
                  pl.BlockSpec((_CHUNK, _PR * _PW), lambda b: (0, 0))],
        out_specs=[pl.BlockSpec((1, _C, _C), lambda b: (b, 0, 0)),
                   pl.BlockSpec((1, _C, _PW * _PW), lambda b: (b, 0, 0))],
        out_shape=[jax.ShapeDtypeStruct((_B, _C, _C), jnp.float32),
                   jax.ShapeDtypeStruct((_B, _C, _PW * _PW), jnp.float32)],
        compiler_params=pltpu.CompilerParams(
            dimension_semantics=("parallel",)),
    )(x3, x3, pmat)

    s_col = (xp3.sum(axis=2) * (_PS * _PS)).reshape(_B, _C, 1)
    dw9 = router_dw_w.reshape(_C, 9)
    pw8 = jnp.zeros((8, _C), jnp.float32).at[:_R].set(router_pw_w)
    fc8 = jnp.zeros((_E, 8), jnp.float32).at[:, :_R].set(router_fc_w)
    fcb = router_fc_b.reshape(_E, 1)

    idx_o, wts_o = pl.pallas_call(
        _router_kernel,
        grid=(_B,),
        in_specs=[pl.BlockSpec((1, _C, _PW * _PW), lambda b: (b, 0, 0)),
                  pl.BlockSpec((_C, 9), lambda b: (0, 0)),
                  pl.BlockSpec((8, _C), lambda b: (0, 0)),
                  pl.BlockSpec((_E, 8), lambda b: (0, 0)),
                  pl.BlockSpec((_E, 1), lambda b: (0, 0))],
        out_specs=[pl.BlockSpec((1, 1, 8), lambda b: (b, 0, 0)),
                   pl.BlockSpec((1, 1, 8), lambda b: (b, 0, 0))],
        out_shape=[jax.ShapeDtypeStruct((_B, 1, 8), jnp.int32),
                   jax.ShapeDtypeStruct((_B, 1, 8), jnp.float32)],
    )(xp3, dw9, pw8, fc8, fcb)

    idx_flat = idx_o[:, 0, :_K].reshape(-1)
    expsb = jnp.stack([expert_gn_scale, expert_gn_bias], axis=-1)
    shsb = jnp.stack([shared_gn_scale, shared_gn_bias], axis=-1)

    grid_spec = pltpu.PrefetchScalarGridSpec(
        num_scalar_prefetch=1,
        grid=(_B, 3),
        in_specs=[
            pl.BlockSpec((1, _C, _C), lambda b, j, ref: (b, 0, 0)),
            pl.BlockSpec((1, _C, 1), lambda b, j, ref: (b, 0, 0)),
            pl.BlockSpec((1, _O, _C),
                         lambda b, j, ref: (ref[b * _K + jnp.maximum(j - 1, 0)], 0, 0)),
            pl.BlockSpec((1, _O, 2),
                         lambda b, j, ref: (ref[b * _K + jnp.maximum(j - 1, 0)], 0, 0)),
            pl.BlockSpec((_O, _C), lambda b, j, ref: (0, 0)),
            pl.BlockSpec((_O, 2), lambda b, j, ref: (0, 0)),
            pl.BlockSpec((1, 1, 8), lambda b, j, ref: (b, 0, 0)),
        ],
        out_specs=[
            pl.BlockSpec((1, 1, _O, _C), lambda b, j, ref: (b, j, 0, 0)),
            pl.BlockSpec((1, 1, _O, 8), lambda b, j, ref: (b, j, 0, 0)),
        ],
    )
    mcat, aff = pl.pallas_call(
        _stats_kernel,
        grid_spec=grid_spec,
        out_shape=[jax.ShapeDtypeStruct((_B, 3, _O, _C), jnp.float32),
                   jax.ShapeDtypeStruct((_B, 3, _O, 8), jnp.float32)],
    )(idx_flat, gram, s_col, expert_w, expsb, shared_w, shsb, wts_o)

    out3 = pl.pallas_call(
        _main_kernel,
        grid=(_B, _NT),
        in_specs=[pl.BlockSpec((1, _C, _T), lambda b, t: (b, 0, t)),
                  pl.BlockSpec((1, 3, _O, _C), lambda b, t: (b, 0, 0, 0)),
                  pl.BlockSpec((1, 3, _O, 8), lambda b, t: (b, 0, 0, 0))],
        out_specs=pl.BlockSpec((1, _O, _T), lambda b, t: (b, 0, t)),
        out_shape=jax.ShapeDtypeStruct((_B, _O, _HW), jnp.float32),
    )(x3, mcat, aff)

    return out3.reshape(_B, _O, _H, _W)
